# baseline (speedup 1.0000x reference)
import functools

import jax
import jax.numpy as jnp
from jax import lax
from jax.experimental import pallas as pl
from jax.experimental.pallas import tpu as pltpu
from jax.experimental.pallas import tpu_sc as plsc

N_NODES = 10000
HID = 64
NC, NS = 2, 16
NW = NC * NS
GROUP = 128
RING = 4
SPLIT_NUM, SPLIT_DEN = 7, 10
N_PAD = 10240
ROWS_PER_TILE = N_PAD // NS


def _lane_bcast(v16, j):
    return lax.gather(
        v16, jnp.full((16, 1), j, jnp.int32),
        lax.GatherDimensionNumbers(
            offset_dims=(), collapsed_slice_dims=(0,), start_index_map=(0,)),
        slice_sizes=(1,),
        mode=lax.GatherScatterMode.PROMISE_IN_BOUNDS)


@functools.cache
def _sc_edge_kernel(ng0, ng1, with_deg):
    mesh = plsc.VectorSubcoreMesh(core_axis_name="c", subcore_axis_name="s")
    out_type = [jax.ShapeDtypeStruct((NC, N_PAD, HID), jnp.float32)]
    if with_deg:
        out_type.append(jax.ShapeDtypeStruct((NC, N_PAD), jnp.float32))
    scratch = [
        pltpu.VMEM((ng0, GROUP), jnp.int32),
        pltpu.VMEM((ng0, GROUP), jnp.int32),
        pltpu.VMEM((ng0, GROUP), jnp.float32),
        pltpu.VMEM((GROUP, HID), jnp.float32),
        pltpu.VMEM_SHARED((N_PAD, HID), jnp.float32),
        pltpu.SemaphoreType.DMA,
    ]
    scratch += [pltpu.VMEM((GROUP, HID), jnp.float32)] * RING
    scratch += [pltpu.SemaphoreType.DMA] * RING
    scratch += [pltpu.SemaphoreType.DMA] * RING
    if with_deg:
        scratch += [
            pltpu.VMEM((GROUP,), jnp.float32),
            pltpu.VMEM((ROWS_PER_TILE,), jnp.float32),
            pltpu.VMEM_SHARED((N_PAD,), jnp.float32),
        ]

    @functools.partial(
        pl.kernel,
        out_type=tuple(out_type),
        mesh=mesh,
        compiler_params=pltpu.CompilerParams(use_tc_tiling_on_sc=False),
        scratch_types=scratch,
    )
    def k(y_hbm, src_hbm, dst_hbm, w_hbm, *rest):
        if with_deg:
            agg_out, deg_out = rest[0], rest[1]
            rest = rest[2:]
        else:
            agg_out = rest[0]
            rest = rest[1:]
        (src_all, dst_all, w_all, zrow, agg_sh, semi) = rest[:6]
        rows_v = rest[6:6 + RING]
        semg = rest[6 + RING:6 + 2 * RING]
        sems = rest[6 + 2 * RING:6 + 3 * RING]
        if with_deg:
            ones_buf, zdeg, deg_sh = rest[6 + 3 * RING:]

        cid = lax.axis_index("c")
        sid = lax.axis_index("s")
        wid = cid * NS + sid
        ng = jnp.where(cid == 0, ng0, ng1)
        zeros16 = jnp.zeros((16,), jnp.float32)
        ones16 = jnp.ones((16,), jnp.float32)

        pltpu.async_copy(src_hbm.at[wid], src_all, semi)
        pltpu.async_copy(dst_hbm.at[wid], dst_all, semi)
        pltpu.async_copy(w_hbm.at[wid], w_all, semi)

        base = sid * ROWS_PER_TILE
        def zrow_body(i, carry):
            for q in range(HID // 16):
                zrow[i, pl.ds(q * 16, 16)] = zeros16
            return carry
        lax.fori_loop(0, GROUP, zrow_body, 0)
        for blk in range(ROWS_PER_TILE // GROUP):
            pltpu.sync_copy(zrow, agg_sh.at[pl.ds(base + blk * GROUP, GROUP)])
        if with_deg:
            def zdeg_body(i, carry):
                zdeg[pl.ds(i * 16, 16)] = zeros16
                return carry
            lax.fori_loop(0, ROWS_PER_TILE // 16, zdeg_body, 0)
            pltpu.sync_copy(zdeg, deg_sh.at[pl.ds(base, ROWS_PER_TILE)])
            def ones_body(i, carry):
                ones_buf[pl.ds(i * 16, 16)] = ones16
                return carry
            lax.fori_loop(0, GROUP // 16, ones_body, 0)

        pltpu.make_async_copy(src_hbm.at[0], src_all, semi).wait()
        pltpu.make_async_copy(dst_hbm.at[0], dst_all, semi).wait()
        pltpu.make_async_copy(w_hbm.at[0], w_all, semi).wait()
        plsc.subcore_barrier()

        def fire_gather(g, b):
            pltpu.async_copy(y_hbm.at[src_all.at[g]], rows_v[b], semg[b])

        def wait_gather(b):
            pltpu.make_async_copy(y_hbm.at[src_all.at[0]], rows_v[b],
                                  semg[b]).wait()

        def fire_scatter(g, b):
            pltpu.async_copy(rows_v[b], agg_sh.at[dst_all.at[g]], sems[b],
                             add=True)
            if with_deg:
                pltpu.async_copy(ones_buf, deg_sh.at[dst_all.at[g]], sems[b],
                                 add=True)

        def wait_scatter(b):
            pltpu.make_async_copy(rows_v[b], agg_sh.at[dst_all.at[0]],
                                  sems[b]).wait()
            if with_deg:
                pltpu.make_async_copy(ones_buf, deg_sh.at[dst_all.at[0]],
                                      sems[b]).wait()

        def scale(g, b):
            @plsc.parallel_loop(0, GROUP // 16, 1, unroll=1)
            def scale_body(i):
                w16 = w_all[g, pl.ds(i * 16, 16)]
                def edge_body(j, carry):
                    e = i * 16 + j
                    wb = _lane_bcast(w16, j)
                    for q in range(HID // 16):
                        rows_v[b][e, pl.ds(q * 16, 16)] = (
                            rows_v[b][e, pl.ds(q * 16, 16)] * wb)
                    return carry
                lax.fori_loop(0, 16, edge_body, 0)

        for b in range(RING):
            fire_gather(b, b)

        def loop_body(step, carry):
            g0 = step * RING
            for b in range(RING):
                g = g0 + b
                @pl.when(jnp.logical_and(g >= 1, g - 1 + RING < ng))
                def _refill():
                    bp = (b - 1) % RING
                    wait_scatter(bp)
                    fire_gather(g - 1 + RING, bp)
                wait_gather(b)
                scale(g, b)
                fire_scatter(g, b)
            return carry
        lax.fori_loop(0, ng // RING, loop_body, 0)

        for b in range(RING):
            wait_scatter(b)
        plsc.subcore_barrier()

        pltpu.sync_copy(agg_sh.at[pl.ds(base, ROWS_PER_TILE)],
                        agg_out.at[cid, pl.ds(base, ROWS_PER_TILE)])
        if with_deg:
            pltpu.sync_copy(deg_sh.at[pl.ds(base, ROWS_PER_TILE)],
                            deg_out.at[cid, pl.ds(base, ROWS_PER_TILE)])

    return k


def _tc_proj_body(x_ref, wl_ref, wr_ref, bl_ref, y_ref, z_ref):
    xv = x_ref[...]
    y_ref[...] = jnp.dot(xv, wl_ref[...], preferred_element_type=jnp.float32)
    z_ref[...] = (jnp.dot(xv, wr_ref[...], preferred_element_type=jnp.float32)
                  + bl_ref[...])


def _tc_mid_body(agg_ref, deg_ref, z0_ref, wl1_ref, wr1_ref, bl1_ref,
                 y1_ref, z1_ref):
    a = agg_ref[0, :N_NODES, :] + agg_ref[1, :N_NODES, :]
    d = deg_ref[0, :N_NODES, :] + deg_ref[1, :N_NODES, :]
    r = 1.0 / jnp.maximum(d, 1.0)
    h = jax.nn.relu(a * r + z0_ref[...])
    y1_ref[...] = jnp.dot(h, wl1_ref[...], preferred_element_type=jnp.float32)
    z1_ref[...] = (jnp.dot(h, wr1_ref[...], preferred_element_type=jnp.float32)
                   + bl1_ref[...])


def _tc_out_body(agg_ref, deg_ref, z1_ref, wlin_ref, blin_ref, out_ref):
    a = agg_ref[0, :N_NODES, :] + agg_ref[1, :N_NODES, :]
    d = deg_ref[0, :N_NODES, :] + deg_ref[1, :N_NODES, :]
    r = 1.0 / jnp.maximum(d, 1.0)
    o = jnp.tanh(a * r + z1_ref[...])
    logits = (jnp.dot(o, wlin_ref[...], preferred_element_type=jnp.float32)
              + blin_ref[...])
    m = jnp.max(logits, axis=1, keepdims=True)
    e = jnp.exp(logits - m)
    out_ref[...] = e / jnp.sum(e, axis=1, keepdims=True)


def kernel(x, edge_index, edge_weight, Wl0, bl0, Wr0, Wl1, bl1, Wr1, Wlin, blin):
    n = x.shape[0]
    n_edges = edge_index.shape[1]
    unit = SPLIT_DEN * RING
    T = unit * ((n_edges + NS * GROUP * unit - 1) // (NS * GROUP * unit))
    ng0 = SPLIT_NUM * T // SPLIT_DEN
    ng1 = T - ng0
    e_pad = NS * T * GROUP
    pad = e_pad - n_edges

    def shard(a):
        cut = NS * ng0 * GROUP
        h0 = a[:cut].reshape(NS, ng0, GROUP)
        h1 = a[cut:].reshape(NS, ng1, GROUP)
        h1 = jnp.pad(h1, ((0, 0), (0, ng0 - ng1), (0, 0)))
        return jnp.concatenate([h0, h1], axis=0)

    src = edge_index[0].astype(jnp.int32)
    dst = edge_index[1].astype(jnp.int32)
    srcp = shard(jnp.pad(src, (0, pad)))
    dstp = shard(jnp.pad(dst, (0, pad), constant_values=N_PAD - 1))
    wp = shard(jnp.pad(edge_weight.astype(jnp.float32), (0, pad)))

    sds = jax.ShapeDtypeStruct
    y0, z0 = pl.pallas_call(
        _tc_proj_body,
        out_shape=[sds((n, HID), jnp.float32), sds((n, HID), jnp.float32)],
    )(x, Wl0, Wr0, bl0.reshape(1, HID))

    agg0, deg = _sc_edge_kernel(ng0, ng1, True)(y0, srcp, dstp, wp)
    deg3 = deg.reshape(NC, N_PAD, 1)

    y1, z1 = pl.pallas_call(
        _tc_mid_body,
        out_shape=[sds((n, HID), jnp.float32), sds((n, HID), jnp.float32)],
    )(agg0, deg3, z0, Wl1, Wr1, bl1.reshape(1, HID))

    agg1, _ = _sc_edge_kernel(ng0, ng1, True)(y1, srcp, dstp, wp)

    out = pl.pallas_call(
        _tc_out_body,
        out_shape=sds((n, HID), jnp.float32),
    )(agg1, deg3, z1, Wlin, blin.reshape(1, HID))
    return out


# speedup vs baseline: 1.0208x; 1.0208x over previous
import functools

import jax
import jax.numpy as jnp
from jax import lax
from jax.experimental import pallas as pl
from jax.experimental.pallas import tpu as pltpu
from jax.experimental.pallas import tpu_sc as plsc

N_NODES = 10000
HID = 64
NC, NS = 2, 16
NW = NC * NS
GROUP = 128
RING = 4
SPLIT_NUM, SPLIT_DEN = 3, 4
N_PAD = 10240
ROWS_PER_TILE = N_PAD // NS


def _lane_bcast(v16, j):
    return lax.gather(
        v16, jnp.full((16, 1), j, jnp.int32),
        lax.GatherDimensionNumbers(
            offset_dims=(), collapsed_slice_dims=(0,), start_index_map=(0,)),
        slice_sizes=(1,),
        mode=lax.GatherScatterMode.PROMISE_IN_BOUNDS)


@functools.cache
def _sc_edge_kernel(ng0, ng1, with_deg):
    mesh = plsc.VectorSubcoreMesh(core_axis_name="c", subcore_axis_name="s")
    out_type = [jax.ShapeDtypeStruct((NC, N_PAD, HID), jnp.float32)]
    if with_deg:
        out_type.append(jax.ShapeDtypeStruct((NC, N_PAD), jnp.float32))
    scratch = [
        pltpu.VMEM((ng0, GROUP), jnp.int32),
        pltpu.VMEM((ng0, GROUP), jnp.int32),
        pltpu.VMEM((ng0, GROUP), jnp.float32),
        pltpu.VMEM((GROUP, HID), jnp.float32),
        pltpu.VMEM_SHARED((N_PAD, HID), jnp.float32),
        pltpu.SemaphoreType.DMA,
    ]
    scratch += [pltpu.VMEM((GROUP, HID), jnp.float32)] * RING
    scratch += [pltpu.SemaphoreType.DMA] * RING
    scratch += [pltpu.SemaphoreType.DMA] * RING
    if with_deg:
        scratch += [
            pltpu.VMEM((GROUP,), jnp.float32),
            pltpu.VMEM((ROWS_PER_TILE,), jnp.float32),
            pltpu.VMEM_SHARED((N_PAD,), jnp.float32),
        ]

    @functools.partial(
        pl.kernel,
        out_type=tuple(out_type),
        mesh=mesh,
        compiler_params=pltpu.CompilerParams(use_tc_tiling_on_sc=False),
        scratch_types=scratch,
    )
    def k(y_hbm, src_hbm, dst_hbm, w_hbm, *rest):
        if with_deg:
            agg_out, deg_out = rest[0], rest[1]
            rest = rest[2:]
        else:
            agg_out = rest[0]
            rest = rest[1:]
        (src_all, dst_all, w_all, zrow, agg_sh, semi) = rest[:6]
        rows_v = rest[6:6 + RING]
        semg = rest[6 + RING:6 + 2 * RING]
        sems = rest[6 + 2 * RING:6 + 3 * RING]
        if with_deg:
            ones_buf, zdeg, deg_sh = rest[6 + 3 * RING:]

        cid = lax.axis_index("c")
        sid = lax.axis_index("s")
        wid = cid * NS + sid
        ng = jnp.where(cid == 0, ng0, ng1)
        zeros16 = jnp.zeros((16,), jnp.float32)
        ones16 = jnp.ones((16,), jnp.float32)

        pltpu.async_copy(src_hbm.at[wid], src_all, semi)
        pltpu.async_copy(dst_hbm.at[wid], dst_all, semi)
        pltpu.async_copy(w_hbm.at[wid], w_all, semi)

        base = sid * ROWS_PER_TILE
        def zrow_body(i, carry):
            for q in range(HID // 16):
                zrow[i, pl.ds(q * 16, 16)] = zeros16
            return carry
        lax.fori_loop(0, GROUP, zrow_body, 0)
        for blk in range(ROWS_PER_TILE // GROUP):
            pltpu.sync_copy(zrow, agg_sh.at[pl.ds(base + blk * GROUP, GROUP)])
        if with_deg:
            def zdeg_body(i, carry):
                zdeg[pl.ds(i * 16, 16)] = zeros16
                return carry
            lax.fori_loop(0, ROWS_PER_TILE // 16, zdeg_body, 0)
            pltpu.sync_copy(zdeg, deg_sh.at[pl.ds(base, ROWS_PER_TILE)])
            def ones_body(i, carry):
                ones_buf[pl.ds(i * 16, 16)] = ones16
                return carry
            lax.fori_loop(0, GROUP // 16, ones_body, 0)

        pltpu.make_async_copy(src_hbm.at[0], src_all, semi).wait()
        pltpu.make_async_copy(dst_hbm.at[0], dst_all, semi).wait()
        pltpu.make_async_copy(w_hbm.at[0], w_all, semi).wait()
        plsc.subcore_barrier()

        def fire_gather(g, b):
            pltpu.async_copy(y_hbm.at[src_all.at[g]], rows_v[b], semg[b])

        def wait_gather(b):
            pltpu.make_async_copy(y_hbm.at[src_all.at[0]], rows_v[b],
                                  semg[b]).wait()

        def fire_scatter(g, b):
            pltpu.async_copy(rows_v[b], agg_sh.at[dst_all.at[g]], sems[b],
                             add=True)
            if with_deg:
                pltpu.async_copy(ones_buf, deg_sh.at[dst_all.at[g]], sems[b],
                                 add=True)

        def wait_scatter(b):
            pltpu.make_async_copy(rows_v[b], agg_sh.at[dst_all.at[0]],
                                  sems[b]).wait()
            if with_deg:
                pltpu.make_async_copy(ones_buf, deg_sh.at[dst_all.at[0]],
                                      sems[b]).wait()

        def scale(g, b):
            @plsc.parallel_loop(0, GROUP // 16, 1, unroll=1)
            def scale_body(i):
                w16 = w_all[g, pl.ds(i * 16, 16)]
                def edge_body(j, carry):
                    e = i * 16 + j
                    wb = _lane_bcast(w16, j)
                    for q in range(HID // 16):
                        rows_v[b][e, pl.ds(q * 16, 16)] = (
                            rows_v[b][e, pl.ds(q * 16, 16)] * wb)
                    return carry
                lax.fori_loop(0, 16, edge_body, 0)

        for b in range(RING):
            fire_gather(b, b)

        def loop_body(step, carry):
            g0 = step * RING
            for b in range(RING):
                g = g0 + b
                @pl.when(jnp.logical_and(g >= 1, g - 1 + RING < ng))
                def _refill():
                    bp = (b - 1) % RING
                    wait_scatter(bp)
                    fire_gather(g - 1 + RING, bp)
                wait_gather(b)
                scale(g, b)
                fire_scatter(g, b)
            return carry
        lax.fori_loop(0, ng // RING, loop_body, 0)

        for b in range(RING):
            wait_scatter(b)
        plsc.subcore_barrier()

        pltpu.sync_copy(agg_sh.at[pl.ds(base, ROWS_PER_TILE)],
                        agg_out.at[cid, pl.ds(base, ROWS_PER_TILE)])
        if with_deg:
            pltpu.sync_copy(deg_sh.at[pl.ds(base, ROWS_PER_TILE)],
                            deg_out.at[cid, pl.ds(base, ROWS_PER_TILE)])

    return k


def _tc_proj_body(x_ref, wl_ref, wr_ref, bl_ref, y_ref, z_ref):
    xv = x_ref[...]
    y_ref[...] = jnp.dot(xv, wl_ref[...], preferred_element_type=jnp.float32)
    z_ref[...] = (jnp.dot(xv, wr_ref[...], preferred_element_type=jnp.float32)
                  + bl_ref[...])


def _tc_mid_body(agg_ref, deg_ref, z0_ref, wl1_ref, wr1_ref, bl1_ref,
                 y1_ref, z1_ref):
    a = agg_ref[0, :N_NODES, :] + agg_ref[1, :N_NODES, :]
    d = deg_ref[0, :N_NODES, :] + deg_ref[1, :N_NODES, :]
    r = 1.0 / jnp.maximum(d, 1.0)
    h = jax.nn.relu(a * r + z0_ref[...])
    y1_ref[...] = jnp.dot(h, wl1_ref[...], preferred_element_type=jnp.float32)
    z1_ref[...] = (jnp.dot(h, wr1_ref[...], preferred_element_type=jnp.float32)
                   + bl1_ref[...])


def _tc_out_body(agg_ref, deg_ref, z1_ref, wlin_ref, blin_ref, out_ref):
    a = agg_ref[0, :N_NODES, :] + agg_ref[1, :N_NODES, :]
    d = deg_ref[0, :N_NODES, :] + deg_ref[1, :N_NODES, :]
    r = 1.0 / jnp.maximum(d, 1.0)
    o = jnp.tanh(a * r + z1_ref[...])
    logits = (jnp.dot(o, wlin_ref[...], preferred_element_type=jnp.float32)
              + blin_ref[...])
    m = jnp.max(logits, axis=1, keepdims=True)
    e = jnp.exp(logits - m)
    out_ref[...] = e / jnp.sum(e, axis=1, keepdims=True)


def kernel(x, edge_index, edge_weight, Wl0, bl0, Wr0, Wl1, bl1, Wr1, Wlin, blin):
    n = x.shape[0]
    n_edges = edge_index.shape[1]
    unit = SPLIT_DEN * RING
    T = unit * ((n_edges + NS * GROUP * unit - 1) // (NS * GROUP * unit))
    ng0 = SPLIT_NUM * T // SPLIT_DEN
    ng1 = T - ng0
    e_pad = NS * T * GROUP
    pad = e_pad - n_edges

    def shard(a):
        cut = NS * ng0 * GROUP
        h0 = a[:cut].reshape(NS, ng0, GROUP)
        h1 = a[cut:].reshape(NS, ng1, GROUP)
        h1 = jnp.pad(h1, ((0, 0), (0, ng0 - ng1), (0, 0)))
        return jnp.concatenate([h0, h1], axis=0)

    src = edge_index[0].astype(jnp.int32)
    dst = edge_index[1].astype(jnp.int32)
    srcp = shard(jnp.pad(src, (0, pad)))
    dstp = shard(jnp.pad(dst, (0, pad), constant_values=N_PAD - 1))
    wp = shard(jnp.pad(edge_weight.astype(jnp.float32), (0, pad)))

    sds = jax.ShapeDtypeStruct
    y0, z0 = pl.pallas_call(
        _tc_proj_body,
        out_shape=[sds((n, HID), jnp.float32), sds((n, HID), jnp.float32)],
    )(x, Wl0, Wr0, bl0.reshape(1, HID))

    agg0, deg = _sc_edge_kernel(ng0, ng1, True)(y0, srcp, dstp, wp)
    deg3 = deg.reshape(NC, N_PAD, 1)

    y1, z1 = pl.pallas_call(
        _tc_mid_body,
        out_shape=[sds((n, HID), jnp.float32), sds((n, HID), jnp.float32)],
    )(agg0, deg3, z0, Wl1, Wr1, bl1.reshape(1, HID))

    agg1, _ = _sc_edge_kernel(ng0, ng1, True)(y1, srcp, dstp, wp)

    out = pl.pallas_call(
        _tc_out_body,
        out_shape=sds((n, HID), jnp.float32),
    )(agg1, deg3, z1, Wlin, blin.reshape(1, HID))
    return out
